# baseline (speedup 1.0000x reference)
import functools

import jax
import jax.numpy as jnp
from jax import lax
from jax.experimental import pallas as pl
from jax.experimental.pallas import tpu as pltpu
from jax.experimental.pallas import tpu_sc as plsc

N_NODES = 10000
N_EDGES = 320000
D_FEAT = 128
HIDDEN = 16
N_CLASSES = 40

NC = 2
NS = 16
NW = NC * NS

CHUNK = 128
K0 = 80
K1 = 80
ROWS_IDX = NS * (K0 + K1) + (K0 - K1)
E_PAD = ROWS_IDX * CHUNK
N_PAD = 10112
ROWS_T = N_PAD // NS
K_MAX = max(K0, K1)


def _matmul1_body(x_ref, w_ref, o_ref):
    h = jnp.dot(x_ref[...], w_ref[...], preferred_element_type=jnp.float32)
    o_ref[0] = h
    o_ref[1] = h


def _relu_combine_body(p_ref, o_ref):
    r = jnp.maximum(p_ref[0, :N_NODES] + p_ref[1, :N_NODES], 0.0)
    o_ref[0] = r
    o_ref[1] = r


def _combine_matmul2_body(p_ref, w_ref, o_ref):
    h = p_ref[0, :N_NODES] + p_ref[1, :N_NODES]
    o_ref[...] = jnp.dot(h, w_ref[...], preferred_element_type=jnp.float32)


def _make_segsum(d: int):
    mesh = plsc.VectorSubcoreMesh(core_axis_name="c", subcore_axis_name="s")

    @functools.partial(
        pl.kernel,
        out_type=jax.ShapeDtypeStruct((NC, N_PAD, d), jnp.float32),
        mesh=mesh,
        scratch_types=[
            pltpu.VMEM((K_MAX, CHUNK), jnp.int32),
            pltpu.VMEM((K_MAX, CHUNK), jnp.int32),
            pltpu.VMEM((4, CHUNK, d), jnp.float32),
            pltpu.VMEM_SHARED((N_PAD, d), jnp.float32),
            pltpu.SemaphoreType.DMA((4,)),
            pltpu.SemaphoreType.DMA((4,)),
        ],
        compiler_params=pltpu.CompilerParams(use_tc_tiling_on_sc=False),
    )
    def segsum(src_hbm, dst_hbm, h_hbm, zeros_hbm, out_hbm,
               sidx, didx, ring, acc, sg, ss):
        c = lax.axis_index("c")
        s = lax.axis_index("s")
        kw = jnp.where(c == 0, K0, K1)

        pltpu.sync_copy(zeros_hbm.at[pl.ds(s * ROWS_T, ROWS_T)],
                        acc.at[pl.ds(s * ROWS_T, ROWS_T)])
        plsc.subcore_barrier()

        base = jnp.where(c == 0, s * K0, NS * K0 + s * K1)
        pltpu.sync_copy(src_hbm.at[pl.ds(base, K_MAX)], sidx)
        pltpu.sync_copy(dst_hbm.at[pl.ds(base, K_MAX)], didx)

        NB = 4
        for b in range(3):
            pltpu.async_copy(h_hbm.at[c].at[sidx.at[b]], ring.at[b], sg.at[b])

        @pl.loop(0, kw, step=NB)
        def _(j):
            for b in range(NB):
                jb = j + b
                pltpu.make_async_copy(h_hbm.at[c].at[sidx.at[jb]],
                                      ring.at[b], sg.at[b]).wait()
                pltpu.async_copy(ring.at[b], acc.at[didx.at[jb]], ss.at[b],
                                 add=True)
                nb = (b + 3) % NB

                @pl.when(jb + 3 < kw)
                def _():
                    @pl.when(jb >= 1)
                    def _():
                        pltpu.make_async_copy(
                            ring.at[nb], acc.at[didx.at[jb]],
                            ss.at[nb]).wait()

                    pltpu.async_copy(h_hbm.at[c].at[sidx.at[jb + 3]],
                                     ring.at[nb], sg.at[nb])

        for b in range(NB):
            pltpu.make_async_copy(ring.at[b], acc.at[didx.at[0]],
                                  ss.at[b]).wait()

        plsc.subcore_barrier()

        pltpu.sync_copy(acc.at[pl.ds(s * ROWS_T, ROWS_T)],
                        out_hbm.at[c, pl.ds(s * ROWS_T, ROWS_T)])

    return segsum


_segsum_h = _make_segsum(HIDDEN)


def kernel(features, edge_index, weight1, weight2):
    src = edge_index[0].astype(jnp.int32)
    dst = edge_index[1].astype(jnp.int32)
    pad = E_PAD - N_EDGES
    dst_pad = N_NODES + (jnp.arange(pad, dtype=jnp.int32) % (N_PAD - N_NODES))
    src2d = jnp.concatenate(
        [src, jnp.zeros((pad,), jnp.int32)]).reshape(ROWS_IDX, CHUNK)
    dst2d = jnp.concatenate([dst, dst_pad]).reshape(ROWS_IDX, CHUNK)
    zeros_h = jnp.zeros((N_PAD, HIDDEN), jnp.float32)

    h1 = pl.pallas_call(
        _matmul1_body,
        out_shape=jax.ShapeDtypeStruct((NC, N_NODES, HIDDEN), jnp.float32),
    )(features, weight1)

    p1 = _segsum_h(src2d, dst2d, h1, zeros_h)

    r = pl.pallas_call(
        _relu_combine_body,
        out_shape=jax.ShapeDtypeStruct((NC, N_NODES, HIDDEN), jnp.float32),
    )(p1)

    p2 = _segsum_h(src2d, dst2d, r, zeros_h)

    out = pl.pallas_call(
        _combine_matmul2_body,
        out_shape=jax.ShapeDtypeStruct((N_NODES, N_CLASSES), jnp.float32),
    )(p2, weight2)
    return out


# speedup vs baseline: 1.6289x; 1.6289x over previous
import functools

import jax
import jax.numpy as jnp
from jax import lax
from jax.experimental import pallas as pl
from jax.experimental.pallas import tpu as pltpu
from jax.experimental.pallas import tpu_sc as plsc

N_NODES = 10000
N_EDGES = 320000
D_FEAT = 128
HIDDEN = 16
N_CLASSES = 40

NC = 2
NS = 16
NW = NC * NS

CHUNK = 128
ROWS_IDX = N_EDGES // CHUNK
K_LO = ROWS_IDX // NW
K_HI = K_LO + 1
N_HI = ROWS_IDX - NW * K_LO
KCEIL = 80
N_PAD = 10112
ROWS_T = N_PAD // NS


def _matmul1_body(x_ref, w_ref, o_ref):
    o_ref[...] = jnp.dot(x_ref[...], w_ref[...],
                         preferred_element_type=jnp.float32)


def _relu_combine_body(p_ref, o_ref):
    o_ref[...] = jnp.maximum(p_ref[0, :N_NODES] + p_ref[1, :N_NODES], 0.0)


def _combine_matmul2_body(p_ref, w_ref, o_ref):
    h = p_ref[0, :N_NODES] + p_ref[1, :N_NODES]
    o_ref[...] = jnp.dot(h, w_ref[...], preferred_element_type=jnp.float32)


def _make_segsum(d: int):
    mesh = plsc.VectorSubcoreMesh(core_axis_name="c", subcore_axis_name="s")

    @functools.partial(
        pl.kernel,
        out_type=jax.ShapeDtypeStruct((NC, N_PAD, d), jnp.float32),
        mesh=mesh,
        scratch_types=[
            pltpu.VMEM((K_HI, CHUNK), jnp.int32),
            pltpu.VMEM((K_HI, CHUNK), jnp.int32),
            pltpu.VMEM((4, CHUNK, d), jnp.float32),
            pltpu.VMEM_SHARED((N_PAD, d), jnp.float32),
            pltpu.SemaphoreType.DMA((4,)),
            pltpu.SemaphoreType.DMA((4,)),
        ],
        compiler_params=pltpu.CompilerParams(use_tc_tiling_on_sc=False),
    )
    def segsum(src_hbm, dst_hbm, h_hbm, zeros_hbm, out_hbm,
               sidx, didx, ring, acc, sg, ss):
        c = lax.axis_index("c")
        s = lax.axis_index("s")
        wid = s * NC + c
        kw = jnp.where(wid < NW - N_HI, K_LO, K_HI)

        pltpu.sync_copy(zeros_hbm.at[pl.ds(s * ROWS_T, ROWS_T)],
                        acc.at[pl.ds(s * ROWS_T, ROWS_T)])
        plsc.subcore_barrier()

        base = jnp.where(wid < NW - N_HI, K_LO * wid,
                         K_LO * (NW - N_HI) + K_HI * (wid - (NW - N_HI)))
        pltpu.sync_copy(src_hbm.at[pl.ds(base, K_HI)], sidx)
        pltpu.sync_copy(dst_hbm.at[pl.ds(base, K_HI)], didx)

        NB = 4
        for b in range(3):
            pltpu.async_copy(h_hbm.at[sidx.at[b]], ring.at[b], sg.at[b])

        @pl.loop(0, KCEIL, step=NB)
        def _(j):
            for b in range(NB):
                jb = j + b

                @pl.when(jb < kw)
                def _():
                    pltpu.make_async_copy(h_hbm.at[sidx.at[jb]],
                                          ring.at[b], sg.at[b]).wait()
                    pltpu.async_copy(ring.at[b], acc.at[didx.at[jb]],
                                     ss.at[b], add=True)
                    nb = (b + 3) % NB

                    @pl.when(jb + 3 < kw)
                    def _():
                        @pl.when(jb >= 1)
                        def _():
                            pltpu.make_async_copy(
                                ring.at[nb], acc.at[didx.at[jb]],
                                ss.at[nb]).wait()

                        pltpu.async_copy(h_hbm.at[sidx.at[jb + 3]],
                                         ring.at[nb], sg.at[nb])

        for b in range(NB):
            pltpu.make_async_copy(ring.at[b], acc.at[didx.at[0]],
                                  ss.at[b]).wait()

        plsc.subcore_barrier()

        pltpu.sync_copy(acc.at[pl.ds(s * ROWS_T, ROWS_T)],
                        out_hbm.at[c, pl.ds(s * ROWS_T, ROWS_T)])

    return segsum


_segsum_h = _make_segsum(HIDDEN)


def kernel(features, edge_index, weight1, weight2):
    src2d = edge_index[0].astype(jnp.int32).reshape(ROWS_IDX, CHUNK)
    dst2d = edge_index[1].astype(jnp.int32).reshape(ROWS_IDX, CHUNK)
    zeros_h = jnp.zeros((N_PAD, HIDDEN), jnp.float32)

    h1 = pl.pallas_call(
        _matmul1_body,
        out_shape=jax.ShapeDtypeStruct((N_NODES, HIDDEN), jnp.float32),
    )(features, weight1)

    p1 = _segsum_h(src2d, dst2d, h1, zeros_h)

    r = pl.pallas_call(
        _relu_combine_body,
        out_shape=jax.ShapeDtypeStruct((N_NODES, HIDDEN), jnp.float32),
    )(p1)

    p2 = _segsum_h(src2d, dst2d, r, zeros_h)

    out = pl.pallas_call(
        _combine_matmul2_body,
        out_shape=jax.ShapeDtypeStruct((N_NODES, N_CLASSES), jnp.float32),
    )(p2, weight2)
    return out


# speedup vs baseline: 1.8357x; 1.1270x over previous
import functools

import jax
import jax.numpy as jnp
from jax import lax
from jax.experimental import pallas as pl
from jax.experimental.pallas import tpu as pltpu
from jax.experimental.pallas import tpu_sc as plsc

N_NODES = 10000
N_EDGES = 320000
D_FEAT = 128
HIDDEN = 16
N_CLASSES = 40

NC = 2
NS = 16
NW = NC * NS

CHUNK = 128
ROWS_IDX = N_EDGES // CHUNK
K_LO = ROWS_IDX // NW
K_HI = K_LO + 1
N_HI = ROWS_IDX - NW * K_LO
KCEIL = 80
N_PAD = 10112
ROWS_T = N_PAD // NS


def _matmul1_body(x_ref, w_ref, o_ref):
    o_ref[...] = jnp.dot(x_ref[...], w_ref[...],
                         preferred_element_type=jnp.float32)


def _relu_combine_body(p_ref, o_ref):
    o_ref[...] = jnp.maximum(p_ref[0] + p_ref[1], 0.0)


def _combine_matmul2_body(p_ref, w_ref, o_ref):
    h = p_ref[0, :N_NODES] + p_ref[1, :N_NODES]
    o_ref[...] = jnp.dot(h, w_ref[...], preferred_element_type=jnp.float32)


def _make_segsum(d: int):
    mesh = plsc.VectorSubcoreMesh(core_axis_name="c", subcore_axis_name="s")

    @functools.partial(
        pl.kernel,
        out_type=jax.ShapeDtypeStruct((NC, N_PAD, d), jnp.float32),
        mesh=mesh,
        scratch_types=[
            pltpu.VMEM((K_HI, CHUNK), jnp.int32),
            pltpu.VMEM((K_HI, CHUNK), jnp.int32),
            pltpu.VMEM((4, CHUNK, d), jnp.float32),
            pltpu.VMEM_SHARED((N_PAD, d), jnp.float32),
            pltpu.VMEM_SHARED((N_PAD, d), jnp.float32),
            pltpu.SemaphoreType.DMA((4,)),
            pltpu.SemaphoreType.DMA((4,)),
        ],
        compiler_params=pltpu.CompilerParams(use_tc_tiling_on_sc=False),
    )
    def segsum(src_hbm, dst_hbm, h_hbm, zeros_hbm, out_hbm,
               sidx, didx, ring, acc, hsp, sg, ss):
        c = lax.axis_index("c")
        s = lax.axis_index("s")
        wid = s * NC + c
        kw = jnp.where(wid < NW - N_HI, K_LO, K_HI)

        pltpu.sync_copy(zeros_hbm.at[pl.ds(s * ROWS_T, ROWS_T)],
                        acc.at[pl.ds(s * ROWS_T, ROWS_T)])
        pltpu.sync_copy(h_hbm.at[pl.ds(s * ROWS_T, ROWS_T)],
                        hsp.at[pl.ds(s * ROWS_T, ROWS_T)])
        plsc.subcore_barrier()

        base = jnp.where(wid < NW - N_HI, K_LO * wid,
                         K_LO * (NW - N_HI) + K_HI * (wid - (NW - N_HI)))
        pltpu.sync_copy(src_hbm.at[pl.ds(base, K_HI)], sidx)
        pltpu.sync_copy(dst_hbm.at[pl.ds(base, K_HI)], didx)

        NB = 4
        for b in range(3):
            pltpu.async_copy(hsp.at[sidx.at[b]], ring.at[b], sg.at[b])

        @pl.loop(0, KCEIL, step=NB)
        def _(j):
            for b in range(NB):
                jb = j + b

                @pl.when(jb < kw)
                def _():
                    pltpu.make_async_copy(hsp.at[sidx.at[jb]],
                                          ring.at[b], sg.at[b]).wait()
                    pltpu.async_copy(ring.at[b], acc.at[didx.at[jb]],
                                     ss.at[b], add=True)
                    nb = (b + 3) % NB

                    @pl.when(jb + 3 < kw)
                    def _():
                        @pl.when(jb >= 1)
                        def _():
                            pltpu.make_async_copy(
                                ring.at[nb], acc.at[didx.at[jb]],
                                ss.at[nb]).wait()

                        pltpu.async_copy(hsp.at[sidx.at[jb + 3]],
                                         ring.at[nb], sg.at[nb])

        for b in range(NB):
            pltpu.make_async_copy(ring.at[b], acc.at[didx.at[0]],
                                  ss.at[b]).wait()

        plsc.subcore_barrier()

        pltpu.sync_copy(acc.at[pl.ds(s * ROWS_T, ROWS_T)],
                        out_hbm.at[c, pl.ds(s * ROWS_T, ROWS_T)])

    return segsum


_segsum_h = _make_segsum(HIDDEN)


def kernel(features, edge_index, weight1, weight2):
    src2d = edge_index[0].astype(jnp.int32).reshape(ROWS_IDX, CHUNK)
    dst2d = edge_index[1].astype(jnp.int32).reshape(ROWS_IDX, CHUNK)
    zeros_h = jnp.zeros((N_PAD, HIDDEN), jnp.float32)

    x_pad = jnp.pad(features, ((0, N_PAD - N_NODES), (0, 0)))

    h1 = pl.pallas_call(
        _matmul1_body,
        out_shape=jax.ShapeDtypeStruct((N_PAD, HIDDEN), jnp.float32),
    )(x_pad, weight1)

    p1 = _segsum_h(src2d, dst2d, h1, zeros_h)

    r = pl.pallas_call(
        _relu_combine_body,
        out_shape=jax.ShapeDtypeStruct((N_PAD, HIDDEN), jnp.float32),
    )(p1)

    p2 = _segsum_h(src2d, dst2d, r, zeros_h)

    out = pl.pallas_call(
        _combine_matmul2_body,
        out_shape=jax.ShapeDtypeStruct((N_NODES, N_CLASSES), jnp.float32),
    )(p2, weight2)
    return out


# speedup vs baseline: 2.0316x; 1.1067x over previous
import functools

import jax
import jax.numpy as jnp
from jax import lax
from jax.experimental import pallas as pl
from jax.experimental.pallas import tpu as pltpu
from jax.experimental.pallas import tpu_sc as plsc

N_NODES = 10000
N_EDGES = 320000
D_FEAT = 128
HIDDEN = 16
N_CLASSES = 40

NC = 2
NS = 16
NW = NC * NS

CHUNK = 128
ROWS_IDX = N_EDGES // CHUNK
K_LO = ROWS_IDX // NW
K_HI = K_LO + 1
N_HI = ROWS_IDX - NW * K_LO
KCEIL = 80
N_PAD = 10112
ROWS_T = N_PAD // NS


def _matmul1_body(x_ref, w_ref, o_ref):
    o_ref[...] = jnp.dot(x_ref[...], w_ref[...],
                         preferred_element_type=jnp.float32)


def _combine_matmul2_body(p_ref, w_ref, o_ref):
    h = p_ref[0, :N_NODES] + p_ref[1, :N_NODES]
    o_ref[...] = jnp.dot(h, w_ref[...], preferred_element_type=jnp.float32)


def _make_segsum(d: int, combine: bool):
    mesh = plsc.VectorSubcoreMesh(core_axis_name="c", subcore_axis_name="s")
    scratch = [
        pltpu.VMEM((K_HI, CHUNK), jnp.int32),
        pltpu.VMEM((K_HI, CHUNK), jnp.int32),
        pltpu.VMEM((4, CHUNK, d), jnp.float32),
        pltpu.VMEM_SHARED((N_PAD, d), jnp.float32),
        pltpu.VMEM_SHARED((N_PAD, d), jnp.float32),
        pltpu.SemaphoreType.DMA((4,)),
        pltpu.SemaphoreType.DMA((4,)),
    ]
    if combine:
        scratch += [
            pltpu.VMEM((ROWS_T, d), jnp.float32),
            pltpu.VMEM((ROWS_T, d), jnp.float32),
        ]

    @functools.partial(
        pl.kernel,
        out_type=jax.ShapeDtypeStruct((NC, N_PAD, d), jnp.float32),
        mesh=mesh,
        scratch_types=scratch,
        compiler_params=pltpu.CompilerParams(use_tc_tiling_on_sc=False),
    )
    def segsum(src_hbm, dst_hbm, h_hbm, zeros_hbm, out_hbm,
               sidx, didx, ring, acc, hsp, sg, ss, *tbuf):
        c = lax.axis_index("c")
        s = lax.axis_index("s")
        wid = s * NC + c
        kw = jnp.where(wid < NW - N_HI, K_LO, K_HI)

        pltpu.sync_copy(zeros_hbm.at[pl.ds(s * ROWS_T, ROWS_T)],
                        acc.at[pl.ds(s * ROWS_T, ROWS_T)])
        if combine:
            t0, t1 = tbuf
            pltpu.sync_copy(h_hbm.at[0, pl.ds(s * ROWS_T, ROWS_T)], t0)
            pltpu.sync_copy(h_hbm.at[1, pl.ds(s * ROWS_T, ROWS_T)], t1)

            @pl.loop(0, ROWS_T)
            def _(i):
                t0[i] = jnp.maximum(t0[i] + t1[i], 0.0)

            pltpu.sync_copy(t0, hsp.at[pl.ds(s * ROWS_T, ROWS_T)])
        else:
            pltpu.sync_copy(h_hbm.at[pl.ds(s * ROWS_T, ROWS_T)],
                            hsp.at[pl.ds(s * ROWS_T, ROWS_T)])
        plsc.subcore_barrier()

        base = jnp.where(wid < NW - N_HI, K_LO * wid,
                         K_LO * (NW - N_HI) + K_HI * (wid - (NW - N_HI)))
        pltpu.sync_copy(src_hbm.at[pl.ds(base, K_HI)], sidx)
        pltpu.sync_copy(dst_hbm.at[pl.ds(base, K_HI)], didx)

        NB = 4
        for b in range(3):
            pltpu.async_copy(hsp.at[sidx.at[b]], ring.at[b], sg.at[b])

        @pl.loop(0, KCEIL, step=NB)
        def _(j):
            for b in range(NB):
                jb = j + b

                @pl.when(jb < kw)
                def _():
                    pltpu.make_async_copy(hsp.at[sidx.at[jb]],
                                          ring.at[b], sg.at[b]).wait()
                    pltpu.async_copy(ring.at[b], acc.at[didx.at[jb]],
                                     ss.at[b], add=True)
                    nb = (b + 3) % NB

                    @pl.when(jb + 3 < kw)
                    def _():
                        @pl.when(jb >= 1)
                        def _():
                            pltpu.make_async_copy(
                                ring.at[nb], acc.at[didx.at[jb]],
                                ss.at[nb]).wait()

                        pltpu.async_copy(hsp.at[sidx.at[jb + 3]],
                                         ring.at[nb], sg.at[nb])

        for b in range(NB):
            pltpu.make_async_copy(ring.at[b], acc.at[didx.at[0]],
                                  ss.at[b]).wait()

        plsc.subcore_barrier()

        pltpu.sync_copy(acc.at[pl.ds(s * ROWS_T, ROWS_T)],
                        out_hbm.at[c, pl.ds(s * ROWS_T, ROWS_T)])

    return segsum


_segsum_h = _make_segsum(HIDDEN, combine=False)
_segsum_hc = _make_segsum(HIDDEN, combine=True)


def kernel(features, edge_index, weight1, weight2):
    src2d = edge_index[0].astype(jnp.int32).reshape(ROWS_IDX, CHUNK)
    dst2d = edge_index[1].astype(jnp.int32).reshape(ROWS_IDX, CHUNK)
    zeros_h = jnp.zeros((N_PAD, HIDDEN), jnp.float32)

    x_pad = jnp.pad(features, ((0, N_PAD - N_NODES), (0, 0)))

    h1 = pl.pallas_call(
        _matmul1_body,
        out_shape=jax.ShapeDtypeStruct((N_PAD, HIDDEN), jnp.float32),
    )(x_pad, weight1)

    p1 = _segsum_h(src2d, dst2d, h1, zeros_h)

    p2 = _segsum_hc(src2d, dst2d, p1, zeros_h)

    out = pl.pallas_call(
        _combine_matmul2_body,
        out_shape=jax.ShapeDtypeStruct((N_NODES, N_CLASSES), jnp.float32),
    )(p2, weight2)
    return out


# speedup vs baseline: 2.2882x; 1.1263x over previous
import functools

import jax
import jax.numpy as jnp
from jax import lax
from jax.experimental import pallas as pl
from jax.experimental.pallas import tpu as pltpu
from jax.experimental.pallas import tpu_sc as plsc

N_NODES = 10000
N_EDGES = 320000
D_FEAT = 128
HIDDEN = 16
N_CLASSES = 40

NC = 2
NS = 16
NW = NC * NS

CHUNK = 128
ROWS_IDX = N_EDGES // CHUNK
K_LO = ROWS_IDX // NW
K_HI = K_LO + 1
N_HI = ROWS_IDX - NW * K_LO
KCEIL = 80
N_PAD = 10112
ROWS_T = N_PAD // NS


def _matmul1_body(x_ref, w_ref, o_ref):
    o_ref[:N_NODES] = jnp.dot(x_ref[...], w_ref[...],
                              preferred_element_type=jnp.float32)


def _combine_matmul2_body(p_ref, w_ref, o_ref):
    h = p_ref[0, :N_NODES] + p_ref[1, :N_NODES]
    o_ref[...] = jnp.dot(h, w_ref[...], preferred_element_type=jnp.float32)


def _make_segsum(d: int, combine: bool):
    mesh = plsc.VectorSubcoreMesh(core_axis_name="c", subcore_axis_name="s")
    scratch = [
        pltpu.VMEM((K_HI * CHUNK,), jnp.int32),
        pltpu.VMEM((K_HI * CHUNK,), jnp.int32),
        pltpu.VMEM((4, CHUNK, d), jnp.float32),
        pltpu.VMEM_SHARED((N_PAD, d), jnp.float32),
        pltpu.VMEM_SHARED((N_PAD, d), jnp.float32),
        pltpu.SemaphoreType.DMA((4,)),
        pltpu.SemaphoreType.DMA((4,)),
    ]
    if combine:
        scratch += [
            pltpu.VMEM((ROWS_T, d), jnp.float32),
            pltpu.VMEM((ROWS_T, d), jnp.float32),
        ]

    @functools.partial(
        pl.kernel,
        out_type=jax.ShapeDtypeStruct((NC, N_PAD, d), jnp.float32),
        mesh=mesh,
        scratch_types=scratch,
        compiler_params=pltpu.CompilerParams(use_tc_tiling_on_sc=False),
    )
    def segsum(edge_hbm, h_hbm, zeros_hbm, out_hbm,
               sidx, didx, ring, acc, hsp, sg, ss, *tbuf):
        c = lax.axis_index("c")
        s = lax.axis_index("s")
        wid = s * NC + c
        kw = jnp.where(wid < NW - N_HI, K_LO, K_HI)

        pltpu.sync_copy(zeros_hbm.at[pl.ds(s * ROWS_T, ROWS_T)],
                        acc.at[pl.ds(s * ROWS_T, ROWS_T)])
        if combine:
            t0, t1 = tbuf
            pltpu.sync_copy(h_hbm.at[0, pl.ds(s * ROWS_T, ROWS_T)], t0)
            pltpu.sync_copy(h_hbm.at[1, pl.ds(s * ROWS_T, ROWS_T)], t1)

            @pl.loop(0, ROWS_T)
            def _(i):
                t0[i] = jnp.maximum(t0[i] + t1[i], 0.0)

            pltpu.sync_copy(t0, hsp.at[pl.ds(s * ROWS_T, ROWS_T)])
        else:
            pltpu.sync_copy(h_hbm.at[pl.ds(s * ROWS_T, ROWS_T)],
                            hsp.at[pl.ds(s * ROWS_T, ROWS_T)])
        plsc.subcore_barrier()

        base = jnp.where(wid < NW - N_HI, K_LO * wid,
                         K_LO * (NW - N_HI) + K_HI * (wid - (NW - N_HI)))
        pltpu.sync_copy(edge_hbm.at[0, pl.ds(base * CHUNK, K_HI * CHUNK)],
                        sidx)
        pltpu.sync_copy(edge_hbm.at[1, pl.ds(base * CHUNK, K_HI * CHUNK)],
                        didx)

        NB = 4
        for b in range(3):
            pltpu.async_copy(hsp.at[sidx.at[pl.ds(b * CHUNK, CHUNK)]],
                             ring.at[b], sg.at[b])

        @pl.loop(0, KCEIL, step=NB)
        def _(j):
            for b in range(NB):
                jb = j + b

                @pl.when(jb < kw)
                def _():
                    dwin = didx.at[pl.ds(jb * CHUNK, CHUNK)]
                    pltpu.make_async_copy(
                        hsp.at[sidx.at[pl.ds(jb * CHUNK, CHUNK)]],
                        ring.at[b], sg.at[b]).wait()
                    pltpu.async_copy(ring.at[b], acc.at[dwin],
                                     ss.at[b], add=True)
                    nb = (b + 3) % NB

                    @pl.when(jb + 3 < kw)
                    def _():
                        @pl.when(jb >= 1)
                        def _():
                            pltpu.make_async_copy(
                                ring.at[nb], acc.at[dwin],
                                ss.at[nb]).wait()

                        pltpu.async_copy(
                            hsp.at[sidx.at[pl.ds((jb + 3) * CHUNK, CHUNK)]],
                            ring.at[nb], sg.at[nb])

        for b in range(NB):
            pltpu.make_async_copy(ring.at[b],
                                  acc.at[didx.at[pl.ds(0, CHUNK)]],
                                  ss.at[b]).wait()

        plsc.subcore_barrier()

        pltpu.sync_copy(acc.at[pl.ds(s * ROWS_T, ROWS_T)],
                        out_hbm.at[c, pl.ds(s * ROWS_T, ROWS_T)])

    return segsum


_segsum_h = _make_segsum(HIDDEN, combine=False)
_segsum_hc = _make_segsum(HIDDEN, combine=True)


def kernel(features, edge_index, weight1, weight2):
    edges = edge_index.astype(jnp.int32)
    zeros_h = jnp.zeros((N_PAD, HIDDEN), jnp.float32)

    h1 = pl.pallas_call(
        _matmul1_body,
        out_shape=jax.ShapeDtypeStruct((N_PAD, HIDDEN), jnp.float32),
    )(features, weight1)

    p1 = _segsum_h(edges, h1, zeros_h)

    p2 = _segsum_hc(edges, p1, zeros_h)

    out = pl.pallas_call(
        _combine_matmul2_body,
        out_shape=jax.ShapeDtypeStruct((N_NODES, N_CLASSES), jnp.float32),
    )(p2, weight2)
    return out


# speedup vs baseline: 2.4689x; 1.0790x over previous
import functools

import jax
import jax.numpy as jnp
from jax import lax
from jax.experimental import pallas as pl
from jax.experimental.pallas import tpu as pltpu
from jax.experimental.pallas import tpu_sc as plsc

N_NODES = 10000
N_EDGES = 320000
D_FEAT = 128
HIDDEN = 16
N_CLASSES = 40

NC = 2
NS = 16
NW = NC * NS

CHUNK = 128
ROWS_IDX = N_EDGES // CHUNK
K_LO = ROWS_IDX // NW
K_HI = K_LO + 1
N_HI = ROWS_IDX - NW * K_LO
KCEIL = 80
N_PAD = 10112
ROWS_T = N_PAD // NS


def _matmul1_body(x_ref, w_ref, o_ref):
    o_ref[:N_NODES] = jnp.dot(x_ref[...], w_ref[...],
                              preferred_element_type=jnp.float32)


def _combine_matmul2_body(p_ref, w_ref, o_ref):
    o_ref[...] = jnp.dot(p_ref[0] + p_ref[1], w_ref[...],
                         preferred_element_type=jnp.float32)


def _make_segsum(d: int, combine: bool):
    mesh = plsc.VectorSubcoreMesh(core_axis_name="c", subcore_axis_name="s")
    scratch = [
        pltpu.VMEM((K_HI * CHUNK,), jnp.int32),
        pltpu.VMEM((K_HI * CHUNK,), jnp.int32),
        pltpu.VMEM((4, CHUNK, d), jnp.float32),
        pltpu.VMEM_SHARED((N_PAD, d), jnp.float32),
        pltpu.VMEM_SHARED((N_PAD, d), jnp.float32),
        pltpu.SemaphoreType.DMA((4,)),
        pltpu.SemaphoreType.DMA((4,)),
    ]
    if combine:
        scratch += [
            pltpu.VMEM((ROWS_T, d), jnp.float32),
            pltpu.VMEM((ROWS_T, d), jnp.float32),
        ]

    @functools.partial(
        pl.kernel,
        out_type=jax.ShapeDtypeStruct((NC, N_PAD, d), jnp.float32),
        mesh=mesh,
        scratch_types=scratch,
        compiler_params=pltpu.CompilerParams(use_tc_tiling_on_sc=False),
    )
    def segsum(edge_hbm, h_hbm, zeros_hbm, out_hbm,
               sidx, didx, ring, acc, hsp, sg, ss, *tbuf):
        c = lax.axis_index("c")
        s = lax.axis_index("s")
        wid = s * NC + c
        kw = jnp.where(wid < NW - N_HI, K_LO, K_HI)

        pltpu.sync_copy(zeros_hbm.at[pl.ds(s * ROWS_T, ROWS_T)],
                        acc.at[pl.ds(s * ROWS_T, ROWS_T)])
        if combine:
            t0, t1 = tbuf
            pltpu.sync_copy(h_hbm.at[0, pl.ds(s * ROWS_T, ROWS_T)], t0)
            pltpu.sync_copy(h_hbm.at[1, pl.ds(s * ROWS_T, ROWS_T)], t1)

            @pl.loop(0, ROWS_T)
            def _(i):
                t0[i] = jnp.maximum(t0[i] + t1[i], 0.0)

            pltpu.sync_copy(t0, hsp.at[pl.ds(s * ROWS_T, ROWS_T)])
        else:
            pltpu.sync_copy(h_hbm.at[pl.ds(s * ROWS_T, ROWS_T)],
                            hsp.at[pl.ds(s * ROWS_T, ROWS_T)])
        plsc.subcore_barrier()

        base = jnp.where(wid < NW - N_HI, K_LO * wid,
                         K_LO * (NW - N_HI) + K_HI * (wid - (NW - N_HI)))
        pltpu.sync_copy(edge_hbm.at[0, pl.ds(base * CHUNK, K_HI * CHUNK)],
                        sidx)
        pltpu.sync_copy(edge_hbm.at[1, pl.ds(base * CHUNK, K_HI * CHUNK)],
                        didx)

        NB = 4
        for b in range(3):
            pltpu.async_copy(hsp.at[sidx.at[pl.ds(b * CHUNK, CHUNK)]],
                             ring.at[b], sg.at[b])

        @pl.loop(0, KCEIL, step=NB)
        def _(j):
            for b in range(NB):
                jb = j + b

                @pl.when(jb < kw)
                def _():
                    dwin = didx.at[pl.ds(jb * CHUNK, CHUNK)]
                    pltpu.make_async_copy(
                        hsp.at[sidx.at[pl.ds(jb * CHUNK, CHUNK)]],
                        ring.at[b], sg.at[b]).wait()
                    pltpu.async_copy(ring.at[b], acc.at[dwin],
                                     ss.at[b], add=True)
                    nb = (b + 3) % NB

                    @pl.when(jb + 3 < kw)
                    def _():
                        @pl.when(jb >= 1)
                        def _():
                            pltpu.make_async_copy(
                                ring.at[nb], acc.at[dwin],
                                ss.at[nb]).wait()

                        pltpu.async_copy(
                            hsp.at[sidx.at[pl.ds((jb + 3) * CHUNK, CHUNK)]],
                            ring.at[nb], sg.at[nb])

        for b in range(NB):
            pltpu.make_async_copy(ring.at[b],
                                  acc.at[didx.at[pl.ds(0, CHUNK)]],
                                  ss.at[b]).wait()

        plsc.subcore_barrier()

        pltpu.sync_copy(acc.at[pl.ds(s * ROWS_T, ROWS_T)],
                        out_hbm.at[c, pl.ds(s * ROWS_T, ROWS_T)])

    return segsum


_segsum_h = _make_segsum(HIDDEN, combine=False)
_segsum_hc = _make_segsum(HIDDEN, combine=True)


def kernel(features, edge_index, weight1, weight2):
    edges = edge_index.astype(jnp.int32)
    zeros_h = jnp.zeros((N_PAD, HIDDEN), jnp.float32)

    h1 = pl.pallas_call(
        _matmul1_body,
        out_shape=jax.ShapeDtypeStruct((N_PAD, HIDDEN), jnp.float32),
    )(features, weight1)

    p1 = _segsum_h(edges, h1, zeros_h)

    p2 = _segsum_hc(edges, p1, zeros_h)

    w2s = jnp.kron(jnp.eye(8, dtype=jnp.float32), weight2)
    t = pl.pallas_call(
        _combine_matmul2_body,
        out_shape=jax.ShapeDtypeStruct((N_PAD // 8, 8 * N_CLASSES),
                                       jnp.float32),
    )(p2.reshape(NC, N_PAD * HIDDEN // 128, 128), w2s)
    return t.reshape(N_PAD, N_CLASSES)[:N_NODES]
